# baseline (speedup 1.0000x reference)
import jax
import jax.numpy as jnp
from jax.experimental import pallas as pl

N = 50000
E = 800000
K = 8


def _embed_body(x_ref, wh_ref, bh_ref, o_ref):
    o_ref[...] = jnp.dot(x_ref[...], wh_ref[...], preferred_element_type=jnp.float32) + bh_ref[...]


def _edge_body(ea_ref, we_ref, be_ref, o_ref):
    o_ref[...] = ea_ref[...] * we_ref[...] + be_ref[...]


def _embed_h(x, Wh, bh):
    n, d = x.shape
    dout = Wh.shape[1]
    blk = 5000
    return pl.pallas_call(
        _embed_body,
        grid=(n // blk,),
        in_specs=[
            pl.BlockSpec((blk, d), lambda i: (i, 0)),
            pl.BlockSpec((d, dout), lambda i: (0, 0)),
            pl.BlockSpec((1, dout), lambda i: (0, 0)),
        ],
        out_specs=pl.BlockSpec((blk, dout), lambda i: (i, 0)),
        out_shape=jax.ShapeDtypeStruct((n, dout), jnp.float32),
    )(x, Wh, bh.reshape(1, dout))


def _embed_e(edge_attr, We, be):
    e = edge_attr.shape[0]
    dout = We.shape[1]
    blk = 8000
    return pl.pallas_call(
        _edge_body,
        grid=(e // blk,),
        in_specs=[
            pl.BlockSpec((blk, 1), lambda i: (i, 0)),
            pl.BlockSpec((1, dout), lambda i: (0, 0)),
            pl.BlockSpec((1, dout), lambda i: (0, 0)),
        ],
        out_specs=pl.BlockSpec((blk, dout), lambda i: (i, 0)),
        out_shape=jax.ShapeDtypeStruct((e, dout), jnp.float32),
    )(edge_attr.reshape(e, 1), We.reshape(1, dout), be.reshape(1, dout))


def _bn(h):
    axes = tuple(range(h.ndim - 1))
    mu = h.mean(axis=axes, keepdims=True)
    var = h.var(axis=axes, keepdims=True)
    return (h - mu) / jnp.sqrt(var + 1e-5)


def _gin(h, src, dst, w1, b1, w2, b2):
    agg = jax.ops.segment_sum(h[src], dst, num_segments=N)
    m = h + agg
    m = m @ w1 + b1
    m = _bn(m)
    m = jax.nn.relu(m)
    return m @ w2 + b2


def _signnet(pe, edge_index, phi_params, rho_params):
    src, dst = edge_index[0], edge_index[1]
    z = pe[..., None]
    z = jnp.where(jnp.isnan(z), 0.0, z)

    def enc(h):
        for (w1, b1, w2, b2) in phi_params:
            h = _gin(h, src, dst, w1, b1, w2, b2)
        return h

    h = enc(z) + enc(-z)
    h = h.reshape(h.shape[0], -1)
    rw1, rb1, rw2, rb2 = rho_params
    h = h @ rw1 + rb1
    h = _bn(h)
    h = jax.nn.relu(h)
    return h @ rw2 + rb2


def kernel(x, edge_index, laplacian_pe, batch, edge_attr, Wh, bh, We, be, phi_params, rho_params):
    h = _embed_h(x.astype(jnp.float32), Wh, bh)
    e = _embed_e(edge_attr.astype(jnp.float32), We, be)
    pos_enc = _signnet(laplacian_pe, edge_index, phi_params, rho_params)
    x_new = jnp.concatenate([h, pos_enc], axis=1)
    return x_new, e, pos_enc


# speedup vs baseline: 100.4043x; 100.4043x over previous
import functools

import jax
import jax.numpy as jnp
from jax import lax
from jax.experimental import pallas as pl
from jax.experimental.pallas import tpu as pltpu
from jax.experimental.pallas import tpu_sc as plsc

N = 50000
E = 800000
K = 8

CH = 128
EP = 802816
NA = 50176


def _seg_kernel(nch, wpad, split_features, tab_hbm, srcq_hbm, dst_hbm, zeros_hbm,
                out_hbm, src_v, dst_v, rows_v, acc_sh, sem):
    c = lax.axis_index("c")
    s = lax.axis_index("s")
    zr = NA // 16
    pltpu.sync_copy(zeros_hbm.at[pl.ds(s * zr, zr)], acc_sh.at[pl.ds(s * zr, zr)])
    plsc.subcore_barrier()
    if split_features:
        base = s * (EP // 16)
        idx_base = c * EP + base
        dst_base = base
    else:
        w = c * 16 + s
        base = w * (EP // 32)
        idx_base = base
        dst_base = base

    def body(j, _):
        off = j * CH
        pltpu.sync_copy(srcq_hbm.at[pl.ds(idx_base + off, CH)], src_v)
        pltpu.sync_copy(dst_hbm.at[pl.ds(dst_base + off, CH)], dst_v)
        pltpu.async_copy(tab_hbm.at[src_v], rows_v, sem).wait()
        pltpu.sync_copy(rows_v, acc_sh.at[dst_v], add=True)
        return _

    lax.fori_loop(0, nch, body, None)
    plsc.subcore_barrier()
    pltpu.sync_copy(acc_sh.at[pl.ds(s * zr, zr)],
                    out_hbm.at[c].at[pl.ds(s * zr, zr)])


def _make_seg(nch, wpad, nrows_tab, split_features):
    mesh = plsc.VectorSubcoreMesh(core_axis_name="c", subcore_axis_name="s")
    return pl.kernel(
        functools.partial(_seg_kernel, nch, wpad, split_features),
        out_type=jax.ShapeDtypeStruct((2, NA, wpad), jnp.float32),
        mesh=mesh,
        scratch_types=[
            pltpu.VMEM((CH,), jnp.int32),
            pltpu.VMEM((CH,), jnp.int32),
            pltpu.VMEM((CH, wpad), jnp.float32),
            pltpu.VMEM_SHARED((NA, wpad), jnp.float32),
            pltpu.SemaphoreType.DMA,
        ],
        compiler_params=pltpu.CompilerParams(use_tc_tiling_on_sc=False),
    )


_seg64 = _make_seg(EP // (16 * CH), 32, 2 * N, True)
_seg8 = _make_seg(EP // (32 * CH), 16, N, False)


def _segsum64(tab2n32, srcq, dst, zeros32):
    out = _seg64(tab2n32, srcq, dst, zeros32)
    out = out[:, :N, :]
    return jnp.transpose(out, (1, 0, 2)).reshape(N, K, 8)


def _segsum8(tabn16, src, dst, zeros16):
    out = _seg8(tabn16, src, dst, zeros16)
    agg = out[0, :N, :8] + out[1, :N, :8]
    return agg.reshape(N, K, 1)


def _to_halves(h):
    t = h.reshape(N, 2, 32)
    return jnp.transpose(t, (1, 0, 2)).reshape(2 * N, 32)


def _embed_body(x_ref, wh_ref, bh_ref, o_ref):
    o_ref[...] = jnp.dot(x_ref[...], wh_ref[...], preferred_element_type=jnp.float32) + bh_ref[...]


def _edge_body(ea_ref, we_ref, be_ref, o_ref):
    o_ref[...] = ea_ref[...] * we_ref[...] + be_ref[...]


def _embed_h(x, Wh, bh):
    n, d = x.shape
    dout = Wh.shape[1]
    blk = 5000
    return pl.pallas_call(
        _embed_body,
        grid=(n // blk,),
        in_specs=[
            pl.BlockSpec((blk, d), lambda i: (i, 0)),
            pl.BlockSpec((d, dout), lambda i: (0, 0)),
            pl.BlockSpec((1, dout), lambda i: (0, 0)),
        ],
        out_specs=pl.BlockSpec((blk, dout), lambda i: (i, 0)),
        out_shape=jax.ShapeDtypeStruct((n, dout), jnp.float32),
    )(x, Wh, bh.reshape(1, dout))


def _embed_e(edge_attr, We, be):
    e = edge_attr.shape[0]
    dout = We.shape[1]
    blk = 8000
    return pl.pallas_call(
        _edge_body,
        grid=(e // blk,),
        in_specs=[
            pl.BlockSpec((blk, 1), lambda i: (i, 0)),
            pl.BlockSpec((1, dout), lambda i: (0, 0)),
            pl.BlockSpec((1, dout), lambda i: (0, 0)),
        ],
        out_specs=pl.BlockSpec((blk, dout), lambda i: (i, 0)),
        out_shape=jax.ShapeDtypeStruct((e, dout), jnp.float32),
    )(edge_attr.reshape(e, 1), We.reshape(1, dout), be.reshape(1, dout))


def _bn(h):
    axes = tuple(range(h.ndim - 1))
    mu = h.mean(axis=axes, keepdims=True)
    var = h.var(axis=axes, keepdims=True)
    return (h - mu) / jnp.sqrt(var + 1e-5)


def _signnet(pe, edge_index, phi_params, rho_params):
    src = edge_index[0].astype(jnp.int32)
    dst = edge_index[1].astype(jnp.int32)
    pad = EP - E
    pad_dst = N + (jnp.arange(pad, dtype=jnp.int32) % CH)
    src_p = jnp.concatenate([src, jnp.zeros((pad,), jnp.int32)])
    dst_p = jnp.concatenate([dst, pad_dst])
    srcq = jnp.concatenate([src_p, src_p + N])
    zeros32 = jnp.zeros((NA, 32), jnp.float32)
    zeros16 = jnp.zeros((NA, 16), jnp.float32)

    z = pe[..., None]
    z = jnp.where(jnp.isnan(z), 0.0, z)

    (w11, b11, w12, b12), l2, l3 = phi_params

    ztab = jnp.concatenate([z.reshape(N, K), jnp.zeros((N, K), jnp.float32)], axis=1)
    agg1 = _segsum8(ztab, src_p, dst_p, zeros16)
    m = z + agg1
    u = _bn(m @ w11 + b11)
    hp = jax.nn.relu(u) @ w12 + b12
    hm = jax.nn.relu(-u) @ w12 + b12

    def gin(h, w1, b1, w2, b2):
        agg = _segsum64(_to_halves(h), srcq, dst_p, zeros32)
        m = h + agg
        m = _bn(m @ w1 + b1)
        return jax.nn.relu(m) @ w2 + b2

    for (w1, b1, w2, b2) in (l2, l3):
        hp = gin(hp, w1, b1, w2, b2)
        hm = gin(hm, w1, b1, w2, b2)

    h = hp + hm
    h = h.reshape(N, -1)
    rw1, rb1, rw2, rb2 = rho_params
    h = h @ rw1 + rb1
    h = _bn(h)
    h = jax.nn.relu(h)
    return h @ rw2 + rb2


def kernel(x, edge_index, laplacian_pe, batch, edge_attr, Wh, bh, We, be, phi_params, rho_params):
    h = _embed_h(x.astype(jnp.float32), Wh, bh)
    e = _embed_e(edge_attr.astype(jnp.float32), We, be)
    pos_enc = _signnet(laplacian_pe, edge_index, phi_params, rho_params)
    x_new = jnp.concatenate([h, pos_enc], axis=1)
    return x_new, e, pos_enc


# speedup vs baseline: 208.7970x; 2.0796x over previous
import functools

import jax
import jax.numpy as jnp
from jax import lax
from jax.experimental import pallas as pl
from jax.experimental.pallas import tpu as pltpu
from jax.experimental.pallas import tpu_sc as plsc

N = 50000
E = 800000
K = 8

CH = 128
EP = 802816
NA = 50176


def _seg_kernel(nch, wpad, split_features, tab_hbm, idxq_hbm, zeros_hbm,
                out_hbm, d0, d1, d2, d3, r0, r1, r2, r3, acc_sh,
                i0, i1, i2, i3, g0, g1, g2, g3, s0, s1, s2, s3):
    sd = [d0, d1, d2, d3]
    rows = [r0, r1, r2, r3]
    isem = [i0, i1, i2, i3]
    gsem = [g0, g1, g2, g3]
    ssem = [s0, s1, s2, s3]
    c = lax.axis_index("c")
    s = lax.axis_index("s")
    zr = NA // 16
    pltpu.sync_copy(zeros_hbm.at[pl.ds(s * zr, zr)], acc_sh.at[pl.ds(s * zr, zr)])
    plsc.subcore_barrier()
    if split_features:
        cb = (c * EP + s * (EP // 16)) // CH
    else:
        cb = ((c * 16 + s) * (EP // 32)) // CH

    def fire_idx(jj, b):
        pltpu.async_copy(idxq_hbm.at[cb + jj], sd[b], isem[b])

    def wait_idx(jj, b):
        pltpu.make_async_copy(idxq_hbm.at[cb + jj], sd[b], isem[b]).wait()

    def fire_gather(jj, b):
        pltpu.async_copy(tab_hbm.at[sd[b].at[0]], rows[b], gsem[b])

    def wait_gather(jj, b):
        pltpu.make_async_copy(tab_hbm.at[sd[b].at[0]], rows[b], gsem[b]).wait()

    def fire_scatter(jj, b):
        pltpu.async_copy(rows[b], acc_sh.at[sd[b].at[1]], ssem[b], add=True)

    def wait_scatter(jj, b):
        pltpu.make_async_copy(rows[b], acc_sh.at[sd[b].at[1]], ssem[b]).wait()

    def body(i, _):
        for b in range(4):
            jj = i * 4 + b

            @pl.when(i >= 1)
            def _ws():
                wait_scatter(jj - 4, b)
            fire_idx(jj, b)
            b2 = (b + 2) % 4
            b3 = (b + 1) % 4
            if b >= 2:
                wait_idx(jj - 2, b2)
                fire_gather(jj - 2, b2)
            else:
                @pl.when(i >= 1)
                def _wg():
                    wait_idx(jj - 2, b2)
                    fire_gather(jj - 2, b2)
            if b >= 3:
                wait_gather(jj - 3, b3)
                fire_scatter(jj - 3, b3)
            else:
                @pl.when(i >= 1)
                def _wsc():
                    wait_gather(jj - 3, b3)
                    fire_scatter(jj - 3, b3)
        return _

    lax.fori_loop(0, nch // 4, body, None)
    for jj in (nch - 2, nch - 1):
        b = jj % 4
        wait_idx(jj, b)
        fire_gather(jj, b)
    for jj in (nch - 3, nch - 2, nch - 1):
        b = jj % 4
        wait_gather(jj, b)
        fire_scatter(jj, b)
    for jj in range(nch - 4, nch):
        wait_scatter(jj, jj % 4)
    plsc.subcore_barrier()
    pltpu.sync_copy(acc_sh.at[pl.ds(s * zr, zr)],
                    out_hbm.at[c].at[pl.ds(s * zr, zr)])


def _make_seg(nch, wpad, nrows_tab, split_features):
    mesh = plsc.VectorSubcoreMesh(core_axis_name="c", subcore_axis_name="s")
    dma = pltpu.SemaphoreType.DMA
    return pl.kernel(
        functools.partial(_seg_kernel, nch, wpad, split_features),
        out_type=jax.ShapeDtypeStruct((2, NA, wpad), jnp.float32),
        mesh=mesh,
        scratch_types=(
            [pltpu.VMEM((2, CH), jnp.int32)] * 4
            + [pltpu.VMEM((CH, wpad), jnp.float32)] * 4
            + [pltpu.VMEM_SHARED((NA, wpad), jnp.float32)]
            + [dma] * 12
        ),
        compiler_params=pltpu.CompilerParams(use_tc_tiling_on_sc=False),
    )


_seg64 = _make_seg(EP // (16 * CH), 32, 2 * N, True)
_seg8 = _make_seg(EP // (32 * CH), 16, N, False)


def _segsum64(tab2n32, idxq, zeros32):
    out = _seg64(tab2n32, idxq, zeros32)
    out = out[:, :N, :]
    return jnp.transpose(out, (1, 0, 2)).reshape(N, K, 8)


def _segsum8(tabn16, idxq, zeros16):
    out = _seg8(tabn16, idxq, zeros16)
    agg = out[0, :N, :8] + out[1, :N, :8]
    return agg.reshape(N, K, 1)


def _to_halves(h):
    t = h.reshape(N, 2, 32)
    return jnp.transpose(t, (1, 0, 2)).reshape(2 * N, 32)


def _embed_body(x_ref, wh_ref, bh_ref, o_ref):
    o_ref[...] = jnp.dot(x_ref[...], wh_ref[...], preferred_element_type=jnp.float32) + bh_ref[...]


def _edge_body(ea_ref, we_ref, be_ref, o_ref):
    o_ref[...] = ea_ref[...] * we_ref[...] + be_ref[...]


def _embed_h(x, Wh, bh):
    n, d = x.shape
    dout = Wh.shape[1]
    blk = 5000
    return pl.pallas_call(
        _embed_body,
        grid=(n // blk,),
        in_specs=[
            pl.BlockSpec((blk, d), lambda i: (i, 0)),
            pl.BlockSpec((d, dout), lambda i: (0, 0)),
            pl.BlockSpec((1, dout), lambda i: (0, 0)),
        ],
        out_specs=pl.BlockSpec((blk, dout), lambda i: (i, 0)),
        out_shape=jax.ShapeDtypeStruct((n, dout), jnp.float32),
    )(x, Wh, bh.reshape(1, dout))


def _embed_e(edge_attr, We, be):
    e = edge_attr.shape[0]
    dout = We.shape[1]
    blk = 8000
    return pl.pallas_call(
        _edge_body,
        grid=(e // blk,),
        in_specs=[
            pl.BlockSpec((blk, 1), lambda i: (i, 0)),
            pl.BlockSpec((1, dout), lambda i: (0, 0)),
            pl.BlockSpec((1, dout), lambda i: (0, 0)),
        ],
        out_specs=pl.BlockSpec((blk, dout), lambda i: (i, 0)),
        out_shape=jax.ShapeDtypeStruct((e, dout), jnp.float32),
    )(edge_attr.reshape(e, 1), We.reshape(1, dout), be.reshape(1, dout))


def _bn(h):
    axes = tuple(range(h.ndim - 1))
    mu = h.mean(axis=axes, keepdims=True)
    var = h.var(axis=axes, keepdims=True)
    return (h - mu) / jnp.sqrt(var + 1e-5)


def _signnet(pe, edge_index, phi_params, rho_params):
    src = edge_index[0].astype(jnp.int32)
    dst = edge_index[1].astype(jnp.int32)
    pad = EP - E
    pad_dst = N + (jnp.arange(pad, dtype=jnp.int32) % CH)
    src_p = jnp.concatenate([src, jnp.zeros((pad,), jnp.int32)])
    dst_2d = jnp.concatenate([dst, pad_dst]).reshape(EP // CH, CH)
    src_2d = src_p.reshape(EP // CH, CH)
    idx8 = jnp.stack([src_2d, dst_2d], axis=1)
    srcq_2d = jnp.stack([src_2d, src_2d + N])
    dstb_2d = jnp.broadcast_to(dst_2d, (2, EP // CH, CH))
    idx64 = jnp.stack([srcq_2d, dstb_2d], axis=2).reshape(2 * EP // CH, 2, CH)
    zeros32 = jnp.zeros((NA, 32), jnp.float32)
    zeros16 = jnp.zeros((NA, 16), jnp.float32)

    z = pe[..., None]
    z = jnp.where(jnp.isnan(z), 0.0, z)

    (w11, b11, w12, b12), l2, l3 = phi_params

    ztab = jnp.concatenate([z.reshape(N, K), jnp.zeros((N, K), jnp.float32)], axis=1)
    agg1 = _segsum8(ztab, idx8, zeros16)
    m = z + agg1
    u = _bn(m @ w11 + b11)
    hp = jax.nn.relu(u) @ w12 + b12
    hm = jax.nn.relu(-u) @ w12 + b12

    def gin(h, w1, b1, w2, b2):
        agg = _segsum64(_to_halves(h), idx64, zeros32)
        m = h + agg
        m = _bn(m @ w1 + b1)
        return jax.nn.relu(m) @ w2 + b2

    for (w1, b1, w2, b2) in (l2, l3):
        hp = gin(hp, w1, b1, w2, b2)
        hm = gin(hm, w1, b1, w2, b2)

    h = hp + hm
    h = h.reshape(N, -1)
    rw1, rb1, rw2, rb2 = rho_params
    h = h @ rw1 + rb1
    h = _bn(h)
    h = jax.nn.relu(h)
    return h @ rw2 + rb2


def kernel(x, edge_index, laplacian_pe, batch, edge_attr, Wh, bh, We, be, phi_params, rho_params):
    h = _embed_h(x.astype(jnp.float32), Wh, bh)
    e = _embed_e(edge_attr.astype(jnp.float32), We, be)
    pos_enc = _signnet(laplacian_pe, edge_index, phi_params, rho_params)
    x_new = jnp.concatenate([h, pos_enc], axis=1)
    return x_new, e, pos_enc
